# Initial kernel scaffold; baseline (speedup 1.0000x reference)
import functools

import jax
import jax.numpy as jnp
from jax import lax
from jax.experimental import pallas as pl
from jax.experimental.pallas import tpu as pltpu
from jax.experimental.pallas import tpu_sc as plsc

_N = 50000
_E = 1600000
_F = 128
_H = 32
_G = 256

_NC = 2
_NS = 16
_NW = _NC * _NS

_CH = 128
_NCHUNK = 391
_EW = _CH * _NCHUNK
_EP = _NW * _EW
_NP = 50048
_BLK = 3128
_NBLK = _NP // _BLK
_ACC = 51200
_ZCH = _ACC // _NS
_OCH = _NP // _NS
_DUMMY = _N

_mesh = plsc.VectorSubcoreMesh(core_axis_name="c", subcore_axis_name="s")


@functools.partial(
    pl.kernel,
    out_type=jax.ShapeDtypeStruct((_NC, _NP), jnp.float32),
    mesh=_mesh,
    scratch_types=[
        pltpu.VMEM((_NCHUNK, _CH), jnp.int32),
        pltpu.VMEM((_CH,), jnp.float32),
        pltpu.VMEM_SHARED((_ACC,), jnp.float32),
        pltpu.SemaphoreType.DMA,
    ],
)
def _degree_pass(dst_hbm, zeros1_hbm, out_hbm, dst_v, ones_v, acc, sem):
    c = lax.axis_index("c")
    s = lax.axis_index("s")
    w = s * _NC + c

    def fill(i, carry):
        ones_v[pl.ds(i * 16, 16)] = jnp.ones((16,), jnp.float32)
        return carry

    lax.fori_loop(0, _CH // 16, fill, 0)
    pltpu.sync_copy(zeros1_hbm, acc.at[pl.ds(s * _ZCH, _ZCH)])
    pltpu.async_copy(dst_hbm.at[w], dst_v, sem).wait()
    plsc.subcore_barrier()

    def body(j, carry):
        pltpu.sync_copy(ones_v, acc.at[dst_v.at[j]], add=True)
        return carry

    lax.fori_loop(0, _NCHUNK, body, 0)
    plsc.subcore_barrier()
    pltpu.sync_copy(acc.at[pl.ds(s * _OCH, _OCH)],
                    out_hbm.at[c, pl.ds(s * _OCH, _OCH)])


@functools.partial(
    pl.kernel,
    out_type=jax.ShapeDtypeStruct((_NC, _NP, _H), jnp.float32),
    mesh=_mesh,
    scratch_types=[
        pltpu.VMEM((_NCHUNK, _CH), jnp.int32),
        pltpu.VMEM((_NCHUNK, _CH), jnp.int32),
        pltpu.VMEM((_CH, _H), jnp.float32),
        pltpu.VMEM_SHARED((_ACC, _H), jnp.float32),
        pltpu.SemaphoreType.DMA,
        pltpu.SemaphoreType.DMA,
    ],
)
def _edge_pass(u_hbm, src_hbm, dst_hbm, zeros2_hbm, out_hbm,
               src_v, dst_v, buf, acc, sem_g, sem_i):
    c = lax.axis_index("c")
    s = lax.axis_index("s")
    w = s * _NC + c

    pltpu.sync_copy(zeros2_hbm, acc.at[pl.ds(s * _ZCH, _ZCH)])
    cp_s = pltpu.async_copy(src_hbm.at[w], src_v, sem_i)
    cp_d = pltpu.async_copy(dst_hbm.at[w], dst_v, sem_i)
    cp_s.wait()
    cp_d.wait()
    plsc.subcore_barrier()

    def body(j, carry):
        pltpu.async_copy(u_hbm.at[src_v.at[j]], buf, sem_g).wait()
        pltpu.sync_copy(buf, acc.at[dst_v.at[j]], add=True)
        return carry

    lax.fori_loop(0, _NCHUNK, body, 0)
    plsc.subcore_barrier()
    pltpu.sync_copy(acc.at[pl.ds(s * _OCH, _OCH)],
                    out_hbm.at[c, pl.ds(s * _OCH, _OCH)])


def _u1_body(x_ref, dpt_ref, win_ref, u1_ref, dinv_ref):
    deg = dpt_ref[:, 0:1] + dpt_ref[:, 1:2] + 1.0
    dinv = lax.rsqrt(deg)
    u1_ref[...] = dinv * jnp.dot(x_ref[...], win_ref[...])
    dinv_ref[...] = dinv


def _stage_u1(x_p, dpT, W_in):
    return pl.pallas_call(
        _u1_body,
        grid=(_NBLK,),
        in_specs=[
            pl.BlockSpec((_BLK, _F), lambda i: (i, 0)),
            pl.BlockSpec((_BLK, 2), lambda i: (i, 0)),
            pl.BlockSpec((_F, _H), lambda i: (0, 0)),
        ],
        out_specs=[
            pl.BlockSpec((_BLK, _H), lambda i: (i, 0)),
            pl.BlockSpec((_BLK, 1), lambda i: (i, 0)),
        ],
        out_shape=[
            jax.ShapeDtypeStruct((_NP, _H), jnp.float32),
            jax.ShapeDtypeStruct((_NP, 1), jnp.float32),
        ],
    )(x_p, dpT, W_in)


def _mid_body(a_ref, u1_ref, dinv_ref, b_ref, w1_ref, u2_ref):
    agg = a_ref[0] + a_ref[1] + u1_ref[...]
    h1 = jax.nn.relu(dinv_ref[...] * agg + b_ref[...])
    u2_ref[...] = dinv_ref[...] * jnp.dot(h1, w1_ref[...])


def _stage_mid(a1, u1, dinv, b_in2, W1):
    return pl.pallas_call(
        _mid_body,
        grid=(_NBLK,),
        in_specs=[
            pl.BlockSpec((_NC, _BLK, _H), lambda i: (0, i, 0)),
            pl.BlockSpec((_BLK, _H), lambda i: (i, 0)),
            pl.BlockSpec((_BLK, 1), lambda i: (i, 0)),
            pl.BlockSpec((1, _H), lambda i: (0, 0)),
            pl.BlockSpec((_H, _H), lambda i: (0, 0)),
        ],
        out_specs=pl.BlockSpec((_BLK, _H), lambda i: (i, 0)),
        out_shape=jax.ShapeDtypeStruct((_NP, _H), jnp.float32),
    )(a1, u1, dinv, b_in2, W1)


def _final_body(a_ref, u2_ref, dinv_ref, b_ref, batch_ref, wout_ref, bout_ref,
                out_ref, sums, cnts):
    i = pl.program_id(0)

    @pl.when(i == 0)
    def _():
        sums[...] = jnp.zeros_like(sums)
        cnts[...] = jnp.zeros_like(cnts)

    agg = a_ref[0] + a_ref[1] + u2_ref[...]
    h2 = jax.nn.relu(dinv_ref[...] * agg + b_ref[...])
    gid = lax.broadcasted_iota(jnp.int32, (1, _G), 1)
    mask = (batch_ref[...] == gid).astype(jnp.float32)
    sums[...] += lax.dot_general(mask, h2, (((0,), (0,)), ((), ())),
                                 preferred_element_type=jnp.float32)
    ones = jnp.ones((_BLK, 1), jnp.float32)
    cnts[...] += lax.dot_general(mask, ones, (((0,), (0,)), ((), ())),
                                 preferred_element_type=jnp.float32)

    @pl.when(i == _NBLK - 1)
    def _():
        pooled = sums[...] / jnp.maximum(cnts[...], 1.0)
        out_ref[...] = jnp.dot(pooled, wout_ref[...]) + bout_ref[...]


def _stage_final(a2, u2, dinv, b12, batch_p, W_out, b_out2):
    return pl.pallas_call(
        _final_body,
        grid=(_NBLK,),
        in_specs=[
            pl.BlockSpec((_NC, _BLK, _H), lambda i: (0, i, 0)),
            pl.BlockSpec((_BLK, _H), lambda i: (i, 0)),
            pl.BlockSpec((_BLK, 1), lambda i: (i, 0)),
            pl.BlockSpec((1, _H), lambda i: (0, 0)),
            pl.BlockSpec((_BLK, 1), lambda i: (i, 0)),
            pl.BlockSpec((_H, 1), lambda i: (0, 0)),
            pl.BlockSpec((1, 1), lambda i: (0, 0)),
        ],
        out_specs=pl.BlockSpec((_G, 1), lambda i: (0, 0)),
        out_shape=jax.ShapeDtypeStruct((_G, 1), jnp.float32),
        scratch_shapes=[
            pltpu.VMEM((_G, _H), jnp.float32),
            pltpu.VMEM((_G, 1), jnp.float32),
        ],
    )(a2, u2, dinv, b12, batch_p, W_out, b_out2)


def kernel(x, edge_index, batch_index, W_in, b_in, W1, b1, W_out, b_out):
    src = edge_index[0]
    dst = edge_index[1]
    pad = _EP - _E
    src_p = jnp.concatenate(
        [src, jnp.zeros((pad,), jnp.int32)]).reshape(_NW, _NCHUNK, _CH)
    dst_p = jnp.concatenate(
        [dst, jnp.full((pad,), _DUMMY, jnp.int32)]).reshape(_NW, _NCHUNK, _CH)
    x_p = jnp.pad(x, ((0, _NP - _N), (0, 0)))
    batch_p = jnp.pad(batch_index, (0, _NP - _N),
                      constant_values=_G).reshape(_NP, 1)
    zeros1 = jnp.zeros((_ZCH,), jnp.float32)
    zeros2 = jnp.zeros((_ZCH, _H), jnp.float32)

    dp = _degree_pass(dst_p, zeros1)
    u1, dinv = _stage_u1(x_p, dp.T, W_in)
    a1 = _edge_pass(u1, src_p, dst_p, zeros2)
    u2 = _stage_mid(a1, u1, dinv, b_in.reshape(1, _H), W1)
    a2 = _edge_pass(u2, src_p, dst_p, zeros2)
    return _stage_final(a2, u2, dinv, b1.reshape(1, _H), batch_p,
                        W_out, b_out.reshape(1, 1))


# speedup vs baseline: 24.3508x; 24.3508x over previous
import functools

import jax
import jax.numpy as jnp
from jax import lax
from jax.experimental import pallas as pl
from jax.experimental.pallas import tpu as pltpu
from jax.experimental.pallas import tpu_sc as plsc

_N = 50000
_E = 1600000
_F = 128
_H = 32
_HH = _H // 2
_G = 256

_NC = 2
_NS = 16
_NW = _NC * _NS

_CH = 128
_GRP = 23
_NGRP = 34
_TCHUNK = _GRP * _NGRP
_ET = _CH * _TCHUNK
_EP = _NS * _ET
_NCHUNK = 391
_NP = 50048
_BLK = 3128
_NBLK = _NP // _BLK
_ACC = 51200
_ZCH = _ACC // _NS
_OCH = _NP // _NS
_DUMMY = _N

_mesh = plsc.VectorSubcoreMesh(core_axis_name="c", subcore_axis_name="s")


@functools.partial(
    pl.kernel,
    out_type=jax.ShapeDtypeStruct((_NC, _NP, _HH), jnp.float32),
    mesh=_mesh,
    compiler_params=pltpu.CompilerParams(use_tc_tiling_on_sc=False),
    scratch_types=[
        pltpu.VMEM((_NCHUNK, _CH), jnp.int32),
        pltpu.VMEM((_CH, _HH), jnp.float32),
        pltpu.VMEM_SHARED((_ACC, _HH), jnp.float32),
        pltpu.SemaphoreType.DMA,
    ],
)
def _degree_pass(dst_hbm, ones_hbm, zeros1_hbm, out_hbm, dst_v, ones_v, acc,
                 sem):
    c = lax.axis_index("c")
    s = lax.axis_index("s")
    w = s * _NC + c

    pltpu.sync_copy(ones_hbm, ones_v)
    pltpu.sync_copy(zeros1_hbm, acc.at[pl.ds(s * _ZCH, _ZCH)])
    pltpu.async_copy(dst_hbm.at[w], dst_v, sem).wait()
    plsc.subcore_barrier()

    def body(j, carry):
        pltpu.sync_copy(ones_v, acc.at[dst_v.at[j]], add=True)
        return carry

    lax.fori_loop(0, _NCHUNK, body, 0)
    plsc.subcore_barrier()
    pltpu.sync_copy(acc.at[pl.ds(s * _OCH, _OCH)],
                    out_hbm.at[c, pl.ds(s * _OCH, _OCH)])


@functools.partial(
    pl.kernel,
    out_type=jax.ShapeDtypeStruct((_NC, _NP, _HH), jnp.float32),
    mesh=_mesh,
    compiler_params=pltpu.CompilerParams(use_tc_tiling_on_sc=False),
    scratch_types=[
        pltpu.VMEM((_GRP, _CH), jnp.int32),
        pltpu.VMEM((_GRP, _CH), jnp.int32),
        pltpu.VMEM((_CH, _HH), jnp.float32),
        pltpu.VMEM_SHARED((_ACC, _HH), jnp.float32),
        pltpu.SemaphoreType.DMA,
        pltpu.SemaphoreType.DMA,
    ],
)
def _edge_pass(u_hbm, src_hbm, dst_hbm, zeros2_hbm, out_hbm,
               src_v, dst_v, buf, acc, sem_g, sem_i):
    c = lax.axis_index("c")
    s = lax.axis_index("s")

    pltpu.sync_copy(zeros2_hbm, acc.at[pl.ds(s * _ZCH, _ZCH)])
    plsc.subcore_barrier()
    u_half = u_hbm.at[c]

    def group(g, carry):
        cp_s = pltpu.async_copy(src_hbm.at[s, g], src_v, sem_i)
        cp_d = pltpu.async_copy(dst_hbm.at[s, g], dst_v, sem_i)
        cp_s.wait()
        cp_d.wait()

        def body(j, carry2):
            pltpu.async_copy(u_half.at[src_v.at[j]], buf, sem_g).wait()
            pltpu.sync_copy(buf, acc.at[dst_v.at[j]], add=True)
            return carry2

        return lax.fori_loop(0, _GRP, body, carry)

    lax.fori_loop(0, _NGRP, group, 0)
    plsc.subcore_barrier()
    pltpu.sync_copy(acc.at[pl.ds(s * _OCH, _OCH)],
                    out_hbm.at[c, pl.ds(s * _OCH, _OCH)])


def _u1_body(x_ref, dpt_ref, win_ref, u1_ref, dinv_ref):
    deg = dpt_ref[0, :, 0:1] + dpt_ref[1, :, 0:1] + 1.0
    dinv = lax.rsqrt(deg)
    u = dinv * jnp.dot(x_ref[...], win_ref[...])
    u1_ref[0] = u[:, :_HH]
    u1_ref[1] = u[:, _HH:]
    dinv_ref[...] = dinv


def _stage_u1(x_p, dpT, W_in):
    return pl.pallas_call(
        _u1_body,
        grid=(_NBLK,),
        in_specs=[
            pl.BlockSpec((_BLK, _F), lambda i: (i, 0)),
            pl.BlockSpec((_NC, _BLK, _HH), lambda i: (0, i, 0)),
            pl.BlockSpec((_F, _H), lambda i: (0, 0)),
        ],
        out_specs=[
            pl.BlockSpec((_NC, _BLK, _HH), lambda i: (0, i, 0)),
            pl.BlockSpec((_BLK, 1), lambda i: (i, 0)),
        ],
        out_shape=[
            jax.ShapeDtypeStruct((_NC, _NP, _HH), jnp.float32),
            jax.ShapeDtypeStruct((_NP, 1), jnp.float32),
        ],
    )(x_p, dpT, W_in)


def _mid_body(a_ref, u1_ref, dinv_ref, b_ref, w1_ref, u2_ref):
    agg = jnp.concatenate(
        [a_ref[0] + u1_ref[0], a_ref[1] + u1_ref[1]], axis=1)
    h1 = jax.nn.relu(dinv_ref[...] * agg + b_ref[...])
    u = dinv_ref[...] * jnp.dot(h1, w1_ref[...])
    u2_ref[0] = u[:, :_HH]
    u2_ref[1] = u[:, _HH:]


def _stage_mid(a1, u1, dinv, b_in2, W1):
    return pl.pallas_call(
        _mid_body,
        grid=(_NBLK,),
        in_specs=[
            pl.BlockSpec((_NC, _BLK, _HH), lambda i: (0, i, 0)),
            pl.BlockSpec((_NC, _BLK, _HH), lambda i: (0, i, 0)),
            pl.BlockSpec((_BLK, 1), lambda i: (i, 0)),
            pl.BlockSpec((1, _H), lambda i: (0, 0)),
            pl.BlockSpec((_H, _H), lambda i: (0, 0)),
        ],
        out_specs=pl.BlockSpec((_NC, _BLK, _HH), lambda i: (0, i, 0)),
        out_shape=jax.ShapeDtypeStruct((_NC, _NP, _HH), jnp.float32),
    )(a1, u1, dinv, b_in2, W1)


def _final_body(a_ref, u2_ref, dinv_ref, b_ref, batch_ref, wout_ref, bout_ref,
                out_ref, sums, cnts):
    i = pl.program_id(0)

    @pl.when(i == 0)
    def _():
        sums[...] = jnp.zeros_like(sums)
        cnts[...] = jnp.zeros_like(cnts)

    agg = jnp.concatenate(
        [a_ref[0] + u2_ref[0], a_ref[1] + u2_ref[1]], axis=1)
    h2 = jax.nn.relu(dinv_ref[...] * agg + b_ref[...])
    gid = lax.broadcasted_iota(jnp.int32, (1, _G), 1)
    mask = (batch_ref[...] == gid).astype(jnp.float32)
    sums[...] += lax.dot_general(mask, h2, (((0,), (0,)), ((), ())),
                                 preferred_element_type=jnp.float32)
    ones = jnp.ones((_BLK, 1), jnp.float32)
    cnts[...] += lax.dot_general(mask, ones, (((0,), (0,)), ((), ())),
                                 preferred_element_type=jnp.float32)

    @pl.when(i == _NBLK - 1)
    def _():
        pooled = sums[...] / jnp.maximum(cnts[...], 1.0)
        out_ref[...] = jnp.dot(pooled, wout_ref[...]) + bout_ref[...]


def _stage_final(a2, u2, dinv, b12, batch_p, W_out, b_out2):
    return pl.pallas_call(
        _final_body,
        grid=(_NBLK,),
        in_specs=[
            pl.BlockSpec((_NC, _BLK, _HH), lambda i: (0, i, 0)),
            pl.BlockSpec((_NC, _BLK, _HH), lambda i: (0, i, 0)),
            pl.BlockSpec((_BLK, 1), lambda i: (i, 0)),
            pl.BlockSpec((1, _H), lambda i: (0, 0)),
            pl.BlockSpec((_BLK, 1), lambda i: (i, 0)),
            pl.BlockSpec((_H, 1), lambda i: (0, 0)),
            pl.BlockSpec((1, 1), lambda i: (0, 0)),
        ],
        out_specs=pl.BlockSpec((_G, 1), lambda i: (0, 0)),
        out_shape=jax.ShapeDtypeStruct((_G, 1), jnp.float32),
        scratch_shapes=[
            pltpu.VMEM((_G, _H), jnp.float32),
            pltpu.VMEM((_G, 1), jnp.float32),
        ],
    )(a2, u2, dinv, b12, batch_p, W_out, b_out2)


def kernel(x, edge_index, batch_index, W_in, b_in, W1, b1, W_out, b_out):
    src = edge_index[0]
    dst = edge_index[1]
    pad = _EP - _E
    src_p = jnp.concatenate(
        [src, jnp.zeros((pad,), jnp.int32)]).reshape(_NS, _NGRP, _GRP, _CH)
    dst_flat = jnp.concatenate([dst, jnp.full((pad,), _DUMMY, jnp.int32)])
    dst_p = dst_flat.reshape(_NS, _NGRP, _GRP, _CH)
    dst_p32 = dst_flat.reshape(_NW, _NCHUNK, _CH)
    x_p = jnp.pad(x, ((0, _NP - _N), (0, 0)))
    batch_p = jnp.pad(batch_index, (0, _NP - _N),
                      constant_values=_G).reshape(_NP, 1)
    ones1 = jnp.ones((_CH, _HH), jnp.float32)
    zeros2 = jnp.zeros((_ZCH, _HH), jnp.float32)

    dp = _degree_pass(dst_p32, ones1, zeros2)
    u1, dinv = _stage_u1(x_p, dp, W_in)
    a1 = _edge_pass(u1, src_p, dst_p, zeros2)
    u2 = _stage_mid(a1, u1, dinv, b_in.reshape(1, _H), W1)
    a2 = _edge_pass(u2, src_p, dst_p, zeros2)
    return _stage_final(a2, u2, dinv, b1.reshape(1, _H), batch_p,
                        W_out, b_out.reshape(1, 1))


# speedup vs baseline: 37.2514x; 1.5298x over previous
import functools

import jax
import jax.numpy as jnp
from jax import lax
from jax.experimental import pallas as pl
from jax.experimental.pallas import tpu as pltpu
from jax.experimental.pallas import tpu_sc as plsc

_N = 50000
_E = 1600000
_F = 128
_H = 32
_HH = _H // 2
_G = 256

_NC = 2
_NS = 16
_NW = _NC * _NS

_CH = 128
_GRP = 24
_NGRP = 33
_K = 4
_TCHUNK = _GRP * _NGRP
_ET = _CH * _TCHUNK
_EP = _NS * _ET
_NCHUNK = _EP // (_NW * _CH)
_NP = 50048
_BLK = 3128
_NBLK = _NP // _BLK
_ACC = 51200
_ZCH = _ACC // _NS
_OCH = _NP // _NS
_DUMMY = _N

_mesh = plsc.VectorSubcoreMesh(core_axis_name="c", subcore_axis_name="s")


@functools.partial(
    pl.kernel,
    out_type=jax.ShapeDtypeStruct((_NC, _NP, _HH), jnp.float32),
    mesh=_mesh,
    compiler_params=pltpu.CompilerParams(use_tc_tiling_on_sc=False),
    scratch_types=[
        pltpu.VMEM((_NCHUNK, _CH), jnp.int32),
        pltpu.VMEM((_CH, _HH), jnp.float32),
        pltpu.VMEM_SHARED((_ACC, _HH), jnp.float32),
        pltpu.SemaphoreType.DMA,
    ],
)
def _degree_pass(dst_hbm, ones_hbm, zeros1_hbm, out_hbm, dst_v, ones_v, acc,
                 sem):
    c = lax.axis_index("c")
    s = lax.axis_index("s")
    w = s * _NC + c

    pltpu.sync_copy(ones_hbm, ones_v)
    pltpu.sync_copy(zeros1_hbm, acc.at[pl.ds(s * _ZCH, _ZCH)])
    pltpu.async_copy(dst_hbm.at[w], dst_v, sem).wait()
    plsc.subcore_barrier()

    def body(j, carry):
        pltpu.sync_copy(ones_v, acc.at[dst_v.at[j]], add=True)
        return carry

    lax.fori_loop(0, _NCHUNK, body, 0)
    plsc.subcore_barrier()
    pltpu.sync_copy(acc.at[pl.ds(s * _OCH, _OCH)],
                    out_hbm.at[c, pl.ds(s * _OCH, _OCH)])


@functools.partial(
    pl.kernel,
    out_type=jax.ShapeDtypeStruct((_NC, _NP, _HH), jnp.float32),
    mesh=_mesh,
    compiler_params=pltpu.CompilerParams(use_tc_tiling_on_sc=False),
    scratch_types=[
        pltpu.VMEM((_GRP, _CH), jnp.int32),
        pltpu.VMEM((_GRP, _CH), jnp.int32),
        pltpu.VMEM((_K, _CH, _HH), jnp.float32),
        pltpu.VMEM_SHARED((_ACC, _HH), jnp.float32),
        pltpu.SemaphoreType.DMA,
        pltpu.SemaphoreType.DMA,
    ],
)
def _edge_pass(u_hbm, src_hbm, dst_hbm, zeros2_hbm, out_hbm,
               src_v, dst_v, bufs, acc, sem_g, sem_i):
    c = lax.axis_index("c")
    s = lax.axis_index("s")

    pltpu.sync_copy(zeros2_hbm, acc.at[pl.ds(s * _ZCH, _ZCH)])
    plsc.subcore_barrier()
    u_half = u_hbm.at[c]

    def group(g, carry):
        cp_s = pltpu.async_copy(src_hbm.at[s, g], src_v, sem_i)
        cp_d = pltpu.async_copy(dst_hbm.at[s, g], dst_v, sem_i)
        cp_s.wait()
        cp_d.wait()

        for b in range(_K):
            pltpu.async_copy(u_half.at[src_v.at[b]], bufs.at[b], sem_g)

        def rounds(r, carry2):
            for b in range(_K):
                j = r * _K + b
                pltpu.make_async_copy(
                    u_half.at[src_v.at[b]], bufs.at[b], sem_g).wait()
                pltpu.sync_copy(bufs.at[b], acc.at[dst_v.at[j]], add=True)
                pltpu.async_copy(
                    u_half.at[src_v.at[j + _K]], bufs.at[b], sem_g)
            return carry2

        carry = lax.fori_loop(0, (_GRP - _K) // _K, rounds, carry)
        for b in range(_K):
            j = _GRP - _K + b
            pltpu.make_async_copy(
                u_half.at[src_v.at[b]], bufs.at[b], sem_g).wait()
            pltpu.sync_copy(bufs.at[b], acc.at[dst_v.at[j]], add=True)
        return carry

    lax.fori_loop(0, _NGRP, group, 0)
    plsc.subcore_barrier()
    pltpu.sync_copy(acc.at[pl.ds(s * _OCH, _OCH)],
                    out_hbm.at[c, pl.ds(s * _OCH, _OCH)])


def _u1_body(x_ref, dpt_ref, win_ref, u1_ref, dinv_ref):
    deg = dpt_ref[0, :, 0:1] + dpt_ref[1, :, 0:1] + 1.0
    dinv = lax.rsqrt(deg)
    u = dinv * jnp.dot(x_ref[...], win_ref[...])
    u1_ref[0] = u[:, :_HH]
    u1_ref[1] = u[:, _HH:]
    dinv_ref[...] = dinv


def _stage_u1(x_p, dpT, W_in):
    return pl.pallas_call(
        _u1_body,
        grid=(_NBLK,),
        in_specs=[
            pl.BlockSpec((_BLK, _F), lambda i: (i, 0)),
            pl.BlockSpec((_NC, _BLK, _HH), lambda i: (0, i, 0)),
            pl.BlockSpec((_F, _H), lambda i: (0, 0)),
        ],
        out_specs=[
            pl.BlockSpec((_NC, _BLK, _HH), lambda i: (0, i, 0)),
            pl.BlockSpec((_BLK, 1), lambda i: (i, 0)),
        ],
        out_shape=[
            jax.ShapeDtypeStruct((_NC, _NP, _HH), jnp.float32),
            jax.ShapeDtypeStruct((_NP, 1), jnp.float32),
        ],
    )(x_p, dpT, W_in)


def _mid_body(a_ref, u1_ref, dinv_ref, b_ref, w1_ref, u2_ref):
    agg = jnp.concatenate(
        [a_ref[0] + u1_ref[0], a_ref[1] + u1_ref[1]], axis=1)
    h1 = jax.nn.relu(dinv_ref[...] * agg + b_ref[...])
    u = dinv_ref[...] * jnp.dot(h1, w1_ref[...])
    u2_ref[0] = u[:, :_HH]
    u2_ref[1] = u[:, _HH:]


def _stage_mid(a1, u1, dinv, b_in2, W1):
    return pl.pallas_call(
        _mid_body,
        grid=(_NBLK,),
        in_specs=[
            pl.BlockSpec((_NC, _BLK, _HH), lambda i: (0, i, 0)),
            pl.BlockSpec((_NC, _BLK, _HH), lambda i: (0, i, 0)),
            pl.BlockSpec((_BLK, 1), lambda i: (i, 0)),
            pl.BlockSpec((1, _H), lambda i: (0, 0)),
            pl.BlockSpec((_H, _H), lambda i: (0, 0)),
        ],
        out_specs=pl.BlockSpec((_NC, _BLK, _HH), lambda i: (0, i, 0)),
        out_shape=jax.ShapeDtypeStruct((_NC, _NP, _HH), jnp.float32),
    )(a1, u1, dinv, b_in2, W1)


def _final_body(a_ref, u2_ref, dinv_ref, b_ref, batch_ref, wout_ref, bout_ref,
                out_ref, sums, cnts):
    i = pl.program_id(0)

    @pl.when(i == 0)
    def _():
        sums[...] = jnp.zeros_like(sums)
        cnts[...] = jnp.zeros_like(cnts)

    agg = jnp.concatenate(
        [a_ref[0] + u2_ref[0], a_ref[1] + u2_ref[1]], axis=1)
    h2 = jax.nn.relu(dinv_ref[...] * agg + b_ref[...])
    gid = lax.broadcasted_iota(jnp.int32, (1, _G), 1)
    mask = (batch_ref[...] == gid).astype(jnp.float32)
    sums[...] += lax.dot_general(mask, h2, (((0,), (0,)), ((), ())),
                                 preferred_element_type=jnp.float32)
    ones = jnp.ones((_BLK, 1), jnp.float32)
    cnts[...] += lax.dot_general(mask, ones, (((0,), (0,)), ((), ())),
                                 preferred_element_type=jnp.float32)

    @pl.when(i == _NBLK - 1)
    def _():
        pooled = sums[...] / jnp.maximum(cnts[...], 1.0)
        out_ref[...] = jnp.dot(pooled, wout_ref[...]) + bout_ref[...]


def _stage_final(a2, u2, dinv, b12, batch_p, W_out, b_out2):
    return pl.pallas_call(
        _final_body,
        grid=(_NBLK,),
        in_specs=[
            pl.BlockSpec((_NC, _BLK, _HH), lambda i: (0, i, 0)),
            pl.BlockSpec((_NC, _BLK, _HH), lambda i: (0, i, 0)),
            pl.BlockSpec((_BLK, 1), lambda i: (i, 0)),
            pl.BlockSpec((1, _H), lambda i: (0, 0)),
            pl.BlockSpec((_BLK, 1), lambda i: (i, 0)),
            pl.BlockSpec((_H, 1), lambda i: (0, 0)),
            pl.BlockSpec((1, 1), lambda i: (0, 0)),
        ],
        out_specs=pl.BlockSpec((_G, 1), lambda i: (0, 0)),
        out_shape=jax.ShapeDtypeStruct((_G, 1), jnp.float32),
        scratch_shapes=[
            pltpu.VMEM((_G, _H), jnp.float32),
            pltpu.VMEM((_G, 1), jnp.float32),
        ],
    )(a2, u2, dinv, b12, batch_p, W_out, b_out2)


def kernel(x, edge_index, batch_index, W_in, b_in, W1, b1, W_out, b_out):
    src = edge_index[0]
    dst = edge_index[1]
    pad = _EP - _E
    src_p = jnp.concatenate(
        [src, jnp.zeros((pad,), jnp.int32)]).reshape(_NS, _NGRP, _GRP, _CH)
    dst_flat = jnp.concatenate([dst, jnp.full((pad,), _DUMMY, jnp.int32)])
    dst_p = dst_flat.reshape(_NS, _NGRP, _GRP, _CH)
    dst_p32 = dst_flat.reshape(_NW, _NCHUNK, _CH)
    x_p = jnp.pad(x, ((0, _NP - _N), (0, 0)))
    batch_p = jnp.pad(batch_index, (0, _NP - _N),
                      constant_values=_G).reshape(_NP, 1)
    ones1 = jnp.ones((_CH, _HH), jnp.float32)
    zeros2 = jnp.zeros((_ZCH, _HH), jnp.float32)

    dp = _degree_pass(dst_p32, ones1, zeros2)
    u1, dinv = _stage_u1(x_p, dp, W_in)
    a1 = _edge_pass(u1, src_p, dst_p, zeros2)
    u2 = _stage_mid(a1, u1, dinv, b_in.reshape(1, _H), W1)
    a2 = _edge_pass(u2, src_p, dst_p, zeros2)
    return _stage_final(a2, u2, dinv, b1.reshape(1, _H), batch_p,
                        W_out, b_out.reshape(1, 1))


# speedup vs baseline: 41.0853x; 1.1029x over previous
import functools

import jax
import jax.numpy as jnp
from jax import lax
from jax.experimental import pallas as pl
from jax.experimental.pallas import tpu as pltpu
from jax.experimental.pallas import tpu_sc as plsc

_N = 50000
_E = 1600000
_F = 128
_H = 32
_HH = _H // 2
_G = 256

_NC = 2
_NS = 16
_NW = _NC * _NS

_CH = 128
_GRP = 24
_NGRP = 33
_K = 8
_TCHUNK = _GRP * _NGRP
_ET = _CH * _TCHUNK
_EP = _NS * _ET
_NCHUNK = _EP // (_NW * _CH)
_NP = 50048
_BLK = 3128
_NBLK = _NP // _BLK
_ACC = 51200
_ZCH = _ACC // _NS
_OCH = _NP // _NS
_DUMMY = _N

_mesh = plsc.VectorSubcoreMesh(core_axis_name="c", subcore_axis_name="s")


@functools.partial(
    pl.kernel,
    out_type=jax.ShapeDtypeStruct((_NC, _NP, _HH), jnp.float32),
    mesh=_mesh,
    compiler_params=pltpu.CompilerParams(use_tc_tiling_on_sc=False),
    scratch_types=[
        pltpu.VMEM((_NCHUNK, _CH), jnp.int32),
        pltpu.VMEM((_CH, _HH), jnp.float32),
        pltpu.VMEM_SHARED((_ACC, _HH), jnp.float32),
        pltpu.SemaphoreType.DMA,
    ],
)
def _degree_pass(dst_hbm, ones_hbm, zeros1_hbm, out_hbm, dst_v, ones_v, acc,
                 sem):
    c = lax.axis_index("c")
    s = lax.axis_index("s")
    w = s * _NC + c

    pltpu.sync_copy(ones_hbm, ones_v)
    pltpu.sync_copy(zeros1_hbm, acc.at[pl.ds(s * _ZCH, _ZCH)])
    pltpu.async_copy(dst_hbm.at[w], dst_v, sem).wait()
    plsc.subcore_barrier()

    def body(j, carry):
        pltpu.sync_copy(ones_v, acc.at[dst_v.at[j]], add=True)
        return carry

    lax.fori_loop(0, _NCHUNK, body, 0)
    plsc.subcore_barrier()
    pltpu.sync_copy(acc.at[pl.ds(s * _OCH, _OCH)],
                    out_hbm.at[c, pl.ds(s * _OCH, _OCH)])


@functools.partial(
    pl.kernel,
    out_type=jax.ShapeDtypeStruct((_NC, _NP, _HH), jnp.float32),
    mesh=_mesh,
    compiler_params=pltpu.CompilerParams(use_tc_tiling_on_sc=False),
    scratch_types=[
        pltpu.VMEM((_GRP, _CH), jnp.int32),
        pltpu.VMEM((_GRP, _CH), jnp.int32),
        pltpu.VMEM((_K, _CH, _HH), jnp.float32),
        pltpu.VMEM_SHARED((_ACC, _HH), jnp.float32),
        pltpu.SemaphoreType.DMA,
        pltpu.SemaphoreType.DMA,
    ],
)
def _edge_pass(u_hbm, src_hbm, dst_hbm, zeros2_hbm, out_hbm,
               src_v, dst_v, bufs, acc, sem_g, sem_i):
    c = lax.axis_index("c")
    s = lax.axis_index("s")

    pltpu.sync_copy(zeros2_hbm, acc.at[pl.ds(s * _ZCH, _ZCH)])
    plsc.subcore_barrier()
    u_half = u_hbm.at[c]

    def group(g, carry):
        cp_s = pltpu.async_copy(src_hbm.at[s, g], src_v, sem_i)
        cp_d = pltpu.async_copy(dst_hbm.at[s, g], dst_v, sem_i)
        cp_s.wait()
        cp_d.wait()

        for b in range(_K):
            pltpu.async_copy(u_half.at[src_v.at[b]], bufs.at[b], sem_g)

        def rounds(r, carry2):
            for b in range(_K):
                j = r * _K + b
                pltpu.make_async_copy(
                    u_half.at[src_v.at[b]], bufs.at[b], sem_g).wait()
                pltpu.sync_copy(bufs.at[b], acc.at[dst_v.at[j]], add=True)
                pltpu.async_copy(
                    u_half.at[src_v.at[j + _K]], bufs.at[b], sem_g)
            return carry2

        carry = lax.fori_loop(0, (_GRP - _K) // _K, rounds, carry)
        for b in range(_K):
            j = _GRP - _K + b
            pltpu.make_async_copy(
                u_half.at[src_v.at[b]], bufs.at[b], sem_g).wait()
            pltpu.sync_copy(bufs.at[b], acc.at[dst_v.at[j]], add=True)
        return carry

    lax.fori_loop(0, _NGRP, group, 0)
    plsc.subcore_barrier()
    pltpu.sync_copy(acc.at[pl.ds(s * _OCH, _OCH)],
                    out_hbm.at[c, pl.ds(s * _OCH, _OCH)])


def _u1_body(x_ref, dpt_ref, win_ref, u1_ref, dinv_ref):
    deg = dpt_ref[0, :, 0:1] + dpt_ref[1, :, 0:1] + 1.0
    dinv = lax.rsqrt(deg)
    u = dinv * jnp.dot(x_ref[...], win_ref[...])
    u1_ref[0] = u[:, :_HH]
    u1_ref[1] = u[:, _HH:]
    dinv_ref[...] = dinv


def _stage_u1(x_p, dpT, W_in):
    return pl.pallas_call(
        _u1_body,
        grid=(_NBLK,),
        in_specs=[
            pl.BlockSpec((_BLK, _F), lambda i: (i, 0)),
            pl.BlockSpec((_NC, _BLK, _HH), lambda i: (0, i, 0)),
            pl.BlockSpec((_F, _H), lambda i: (0, 0)),
        ],
        out_specs=[
            pl.BlockSpec((_NC, _BLK, _HH), lambda i: (0, i, 0)),
            pl.BlockSpec((_BLK, 1), lambda i: (i, 0)),
        ],
        out_shape=[
            jax.ShapeDtypeStruct((_NC, _NP, _HH), jnp.float32),
            jax.ShapeDtypeStruct((_NP, 1), jnp.float32),
        ],
    )(x_p, dpT, W_in)


def _mid_body(a_ref, u1_ref, dinv_ref, b_ref, w1_ref, u2_ref):
    agg = jnp.concatenate(
        [a_ref[0] + u1_ref[0], a_ref[1] + u1_ref[1]], axis=1)
    h1 = jax.nn.relu(dinv_ref[...] * agg + b_ref[...])
    u = dinv_ref[...] * jnp.dot(h1, w1_ref[...])
    u2_ref[0] = u[:, :_HH]
    u2_ref[1] = u[:, _HH:]


def _stage_mid(a1, u1, dinv, b_in2, W1):
    return pl.pallas_call(
        _mid_body,
        grid=(_NBLK,),
        in_specs=[
            pl.BlockSpec((_NC, _BLK, _HH), lambda i: (0, i, 0)),
            pl.BlockSpec((_NC, _BLK, _HH), lambda i: (0, i, 0)),
            pl.BlockSpec((_BLK, 1), lambda i: (i, 0)),
            pl.BlockSpec((1, _H), lambda i: (0, 0)),
            pl.BlockSpec((_H, _H), lambda i: (0, 0)),
        ],
        out_specs=pl.BlockSpec((_NC, _BLK, _HH), lambda i: (0, i, 0)),
        out_shape=jax.ShapeDtypeStruct((_NC, _NP, _HH), jnp.float32),
    )(a1, u1, dinv, b_in2, W1)


def _final_body(a_ref, u2_ref, dinv_ref, b_ref, batch_ref, wout_ref, bout_ref,
                out_ref, sums, cnts):
    i = pl.program_id(0)

    @pl.when(i == 0)
    def _():
        sums[...] = jnp.zeros_like(sums)
        cnts[...] = jnp.zeros_like(cnts)

    agg = jnp.concatenate(
        [a_ref[0] + u2_ref[0], a_ref[1] + u2_ref[1]], axis=1)
    h2 = jax.nn.relu(dinv_ref[...] * agg + b_ref[...])
    gid = lax.broadcasted_iota(jnp.int32, (1, _G), 1)
    mask = (batch_ref[...] == gid).astype(jnp.float32)
    sums[...] += lax.dot_general(mask, h2, (((0,), (0,)), ((), ())),
                                 preferred_element_type=jnp.float32)
    ones = jnp.ones((_BLK, 1), jnp.float32)
    cnts[...] += lax.dot_general(mask, ones, (((0,), (0,)), ((), ())),
                                 preferred_element_type=jnp.float32)

    @pl.when(i == _NBLK - 1)
    def _():
        pooled = sums[...] / jnp.maximum(cnts[...], 1.0)
        out_ref[...] = jnp.dot(pooled, wout_ref[...]) + bout_ref[...]


def _stage_final(a2, u2, dinv, b12, batch_p, W_out, b_out2):
    return pl.pallas_call(
        _final_body,
        grid=(_NBLK,),
        in_specs=[
            pl.BlockSpec((_NC, _BLK, _HH), lambda i: (0, i, 0)),
            pl.BlockSpec((_NC, _BLK, _HH), lambda i: (0, i, 0)),
            pl.BlockSpec((_BLK, 1), lambda i: (i, 0)),
            pl.BlockSpec((1, _H), lambda i: (0, 0)),
            pl.BlockSpec((_BLK, 1), lambda i: (i, 0)),
            pl.BlockSpec((_H, 1), lambda i: (0, 0)),
            pl.BlockSpec((1, 1), lambda i: (0, 0)),
        ],
        out_specs=pl.BlockSpec((_G, 1), lambda i: (0, 0)),
        out_shape=jax.ShapeDtypeStruct((_G, 1), jnp.float32),
        scratch_shapes=[
            pltpu.VMEM((_G, _H), jnp.float32),
            pltpu.VMEM((_G, 1), jnp.float32),
        ],
    )(a2, u2, dinv, b12, batch_p, W_out, b_out2)


def kernel(x, edge_index, batch_index, W_in, b_in, W1, b1, W_out, b_out):
    src = edge_index[0]
    dst = edge_index[1]
    pad = _EP - _E
    src_p = jnp.concatenate(
        [src, jnp.zeros((pad,), jnp.int32)]).reshape(_NS, _NGRP, _GRP, _CH)
    dst_flat = jnp.concatenate([dst, jnp.full((pad,), _DUMMY, jnp.int32)])
    dst_p = dst_flat.reshape(_NS, _NGRP, _GRP, _CH)
    dst_p32 = dst_flat.reshape(_NW, _NCHUNK, _CH)
    x_p = jnp.pad(x, ((0, _NP - _N), (0, 0)))
    batch_p = jnp.pad(batch_index, (0, _NP - _N),
                      constant_values=_G).reshape(_NP, 1)
    ones1 = jnp.ones((_CH, _HH), jnp.float32)
    zeros2 = jnp.zeros((_ZCH, _HH), jnp.float32)

    dp = _degree_pass(dst_p32, ones1, zeros2)
    u1, dinv = _stage_u1(x_p, dp, W_in)
    a1 = _edge_pass(u1, src_p, dst_p, zeros2)
    u2 = _stage_mid(a1, u1, dinv, b_in.reshape(1, _H), W1)
    a2 = _edge_pass(u2, src_p, dst_p, zeros2)
    return _stage_final(a2, u2, dinv, b1.reshape(1, _H), batch_p,
                        W_out, b_out.reshape(1, 1))


# speedup vs baseline: 42.7538x; 1.0406x over previous
import functools

import jax
import jax.numpy as jnp
from jax import lax
from jax.experimental import pallas as pl
from jax.experimental.pallas import tpu as pltpu
from jax.experimental.pallas import tpu_sc as plsc

_N = 50000
_E = 1600000
_F = 128
_H = 32
_HH = _H // 2
_G = 256

_NC = 2
_NS = 16
_NW = _NC * _NS

_CH = 128
_GRP = 24
_NGRP = 33
_K = 8
_TCHUNK = _GRP * _NGRP
_ET = _CH * _TCHUNK
_EP = _NS * _ET
_NCHUNK = _EP // (_NW * _CH)
_NP = 50048
_BLK = 3128
_NBLK = _NP // _BLK
_ACC = 51200
_ZCH = _ACC // _NS
_OCH = _NP // _NS
_DUMMY = _N

_mesh = plsc.VectorSubcoreMesh(core_axis_name="c", subcore_axis_name="s")


@functools.partial(
    pl.kernel,
    out_type=jax.ShapeDtypeStruct((_NC, _NP, _HH), jnp.float32),
    mesh=_mesh,
    compiler_params=pltpu.CompilerParams(use_tc_tiling_on_sc=False),
    scratch_types=[
        pltpu.VMEM((_NCHUNK, _CH), jnp.int32),
        pltpu.VMEM((_CH, _HH), jnp.float32),
        pltpu.VMEM_SHARED((_ACC, _HH), jnp.float32),
        pltpu.SemaphoreType.DMA,
    ],
)
def _degree_pass(dst_hbm, ones_hbm, zeros1_hbm, out_hbm, dst_v, ones_v, acc,
                 sem):
    c = lax.axis_index("c")
    s = lax.axis_index("s")
    w = s * _NC + c

    pltpu.sync_copy(ones_hbm, ones_v)
    pltpu.sync_copy(zeros1_hbm, acc.at[pl.ds(s * _ZCH, _ZCH)])
    pltpu.async_copy(dst_hbm.at[w], dst_v, sem).wait()
    plsc.subcore_barrier()

    def body(j, carry):
        pltpu.sync_copy(ones_v, acc.at[dst_v.at[j]], add=True)
        return carry

    lax.fori_loop(0, _NCHUNK, body, 0)
    plsc.subcore_barrier()
    pltpu.sync_copy(acc.at[pl.ds(s * _OCH, _OCH)],
                    out_hbm.at[c, pl.ds(s * _OCH, _OCH)])


@functools.partial(
    pl.kernel,
    out_type=jax.ShapeDtypeStruct((_NC, _NP, _HH), jnp.float32),
    mesh=_mesh,
    compiler_params=pltpu.CompilerParams(use_tc_tiling_on_sc=False),
    scratch_types=[
        pltpu.VMEM((2, _GRP, _CH), jnp.int32),
        pltpu.VMEM((2, _GRP, _CH), jnp.int32),
        pltpu.VMEM((_K, _CH, _HH), jnp.float32),
        pltpu.VMEM_SHARED((_ACC, _HH), jnp.float32),
        pltpu.SemaphoreType.DMA,
        pltpu.SemaphoreType.DMA,
    ],
)
def _edge_pass(u_hbm, src_hbm, dst_hbm, zeros2_hbm, out_hbm,
               src_v, dst_v, bufs, acc, sem_g, sem_i):
    c = lax.axis_index("c")
    s = lax.axis_index("s")

    pltpu.sync_copy(zeros2_hbm, acc.at[pl.ds(s * _ZCH, _ZCH)])
    plsc.subcore_barrier()
    u_half = u_hbm.at[c]

    pltpu.async_copy(src_hbm.at[s, 0], src_v.at[0], sem_i)
    pltpu.async_copy(dst_hbm.at[s, 0], dst_v.at[0], sem_i)

    def group(g, carry):
        p = lax.rem(g, 2)
        pltpu.make_async_copy(src_hbm.at[s, g], src_v.at[p], sem_i).wait()
        pltpu.make_async_copy(dst_hbm.at[s, g], dst_v.at[p], sem_i).wait()
        gn = jnp.minimum(g + 1, _NGRP - 1)
        pltpu.async_copy(src_hbm.at[s, gn], src_v.at[lax.rem(gn, 2)], sem_i)
        pltpu.async_copy(dst_hbm.at[s, gn], dst_v.at[lax.rem(gn, 2)], sem_i)
        sv = src_v.at[p]
        dv = dst_v.at[p]

        for b in range(_K):
            pltpu.async_copy(u_half.at[sv.at[b]], bufs.at[b], sem_g)

        def rounds(r, carry2):
            for b in range(_K):
                j = r * _K + b
                pltpu.make_async_copy(
                    u_half.at[sv.at[b]], bufs.at[b], sem_g).wait()
                pltpu.sync_copy(bufs.at[b], acc.at[dv.at[j]], add=True)
                pltpu.async_copy(
                    u_half.at[sv.at[j + _K]], bufs.at[b], sem_g)
            return carry2

        carry = lax.fori_loop(0, (_GRP - _K) // _K, rounds, carry)
        for b in range(_K):
            j = _GRP - _K + b
            pltpu.make_async_copy(
                u_half.at[sv.at[b]], bufs.at[b], sem_g).wait()
            pltpu.sync_copy(bufs.at[b], acc.at[dv.at[j]], add=True)
        return carry

    lax.fori_loop(0, _NGRP, group, 0)
    pltpu.make_async_copy(src_hbm.at[s, 0], src_v.at[0], sem_i).wait()
    pltpu.make_async_copy(dst_hbm.at[s, 0], dst_v.at[0], sem_i).wait()
    plsc.subcore_barrier()
    pltpu.sync_copy(acc.at[pl.ds(s * _OCH, _OCH)],
                    out_hbm.at[c, pl.ds(s * _OCH, _OCH)])


def _u1_body(x_ref, dpt_ref, win_ref, u1_ref, dinv_ref):
    deg = dpt_ref[0, :, 0:1] + dpt_ref[1, :, 0:1] + 1.0
    dinv = lax.rsqrt(deg)
    u = dinv * jnp.dot(x_ref[...], win_ref[...])
    u1_ref[0] = u[:, :_HH]
    u1_ref[1] = u[:, _HH:]
    dinv_ref[...] = dinv


def _stage_u1(x_p, dpT, W_in):
    return pl.pallas_call(
        _u1_body,
        grid=(_NBLK,),
        in_specs=[
            pl.BlockSpec((_BLK, _F), lambda i: (i, 0)),
            pl.BlockSpec((_NC, _BLK, _HH), lambda i: (0, i, 0)),
            pl.BlockSpec((_F, _H), lambda i: (0, 0)),
        ],
        out_specs=[
            pl.BlockSpec((_NC, _BLK, _HH), lambda i: (0, i, 0)),
            pl.BlockSpec((_BLK, 1), lambda i: (i, 0)),
        ],
        out_shape=[
            jax.ShapeDtypeStruct((_NC, _NP, _HH), jnp.float32),
            jax.ShapeDtypeStruct((_NP, 1), jnp.float32),
        ],
    )(x_p, dpT, W_in)


def _mid_body(a_ref, u1_ref, dinv_ref, b_ref, w1_ref, u2_ref):
    agg = jnp.concatenate(
        [a_ref[0] + u1_ref[0], a_ref[1] + u1_ref[1]], axis=1)
    h1 = jax.nn.relu(dinv_ref[...] * agg + b_ref[...])
    u = dinv_ref[...] * jnp.dot(h1, w1_ref[...])
    u2_ref[0] = u[:, :_HH]
    u2_ref[1] = u[:, _HH:]


def _stage_mid(a1, u1, dinv, b_in2, W1):
    return pl.pallas_call(
        _mid_body,
        grid=(_NBLK,),
        in_specs=[
            pl.BlockSpec((_NC, _BLK, _HH), lambda i: (0, i, 0)),
            pl.BlockSpec((_NC, _BLK, _HH), lambda i: (0, i, 0)),
            pl.BlockSpec((_BLK, 1), lambda i: (i, 0)),
            pl.BlockSpec((1, _H), lambda i: (0, 0)),
            pl.BlockSpec((_H, _H), lambda i: (0, 0)),
        ],
        out_specs=pl.BlockSpec((_NC, _BLK, _HH), lambda i: (0, i, 0)),
        out_shape=jax.ShapeDtypeStruct((_NC, _NP, _HH), jnp.float32),
    )(a1, u1, dinv, b_in2, W1)


def _final_body(a_ref, u2_ref, dinv_ref, b_ref, batch_ref, wout_ref, bout_ref,
                out_ref, sums, cnts):
    i = pl.program_id(0)

    @pl.when(i == 0)
    def _():
        sums[...] = jnp.zeros_like(sums)
        cnts[...] = jnp.zeros_like(cnts)

    agg = jnp.concatenate(
        [a_ref[0] + u2_ref[0], a_ref[1] + u2_ref[1]], axis=1)
    h2 = jax.nn.relu(dinv_ref[...] * agg + b_ref[...])
    gid = lax.broadcasted_iota(jnp.int32, (1, _G), 1)
    mask = (batch_ref[...] == gid).astype(jnp.float32)
    sums[...] += lax.dot_general(mask, h2, (((0,), (0,)), ((), ())),
                                 preferred_element_type=jnp.float32)
    ones = jnp.ones((_BLK, 1), jnp.float32)
    cnts[...] += lax.dot_general(mask, ones, (((0,), (0,)), ((), ())),
                                 preferred_element_type=jnp.float32)

    @pl.when(i == _NBLK - 1)
    def _():
        pooled = sums[...] / jnp.maximum(cnts[...], 1.0)
        out_ref[...] = jnp.dot(pooled, wout_ref[...]) + bout_ref[...]


def _stage_final(a2, u2, dinv, b12, batch_p, W_out, b_out2):
    return pl.pallas_call(
        _final_body,
        grid=(_NBLK,),
        in_specs=[
            pl.BlockSpec((_NC, _BLK, _HH), lambda i: (0, i, 0)),
            pl.BlockSpec((_NC, _BLK, _HH), lambda i: (0, i, 0)),
            pl.BlockSpec((_BLK, 1), lambda i: (i, 0)),
            pl.BlockSpec((1, _H), lambda i: (0, 0)),
            pl.BlockSpec((_BLK, 1), lambda i: (i, 0)),
            pl.BlockSpec((_H, 1), lambda i: (0, 0)),
            pl.BlockSpec((1, 1), lambda i: (0, 0)),
        ],
        out_specs=pl.BlockSpec((_G, 1), lambda i: (0, 0)),
        out_shape=jax.ShapeDtypeStruct((_G, 1), jnp.float32),
        scratch_shapes=[
            pltpu.VMEM((_G, _H), jnp.float32),
            pltpu.VMEM((_G, 1), jnp.float32),
        ],
    )(a2, u2, dinv, b12, batch_p, W_out, b_out2)


def kernel(x, edge_index, batch_index, W_in, b_in, W1, b1, W_out, b_out):
    src = edge_index[0]
    dst = edge_index[1]
    pad = _EP - _E
    src_p = jnp.concatenate(
        [src, jnp.zeros((pad,), jnp.int32)]).reshape(_NS, _NGRP, _GRP, _CH)
    dst_flat = jnp.concatenate([dst, jnp.full((pad,), _DUMMY, jnp.int32)])
    dst_p = dst_flat.reshape(_NS, _NGRP, _GRP, _CH)
    dst_p32 = dst_flat.reshape(_NW, _NCHUNK, _CH)
    x_p = jnp.pad(x, ((0, _NP - _N), (0, 0)))
    batch_p = jnp.pad(batch_index, (0, _NP - _N),
                      constant_values=_G).reshape(_NP, 1)
    ones1 = jnp.ones((_CH, _HH), jnp.float32)
    zeros2 = jnp.zeros((_ZCH, _HH), jnp.float32)

    dp = _degree_pass(dst_p32, ones1, zeros2)
    u1, dinv = _stage_u1(x_p, dp, W_in)
    a1 = _edge_pass(u1, src_p, dst_p, zeros2)
    u2 = _stage_mid(a1, u1, dinv, b_in.reshape(1, _H), W1)
    a2 = _edge_pass(u2, src_p, dst_p, zeros2)
    return _stage_final(a2, u2, dinv, b1.reshape(1, _H), batch_p,
                        W_out, b_out.reshape(1, 1))
